# Initial kernel scaffold; baseline (speedup 1.0000x reference)
import jax
import jax.numpy as jnp
from jax.experimental import pallas as pl


def kernel(x, edge_index, edge_mask, Wr0, Wroot0, broot0, g0, b0, Wr1, Wroot1, broot1, g1, b1):
    raise NotImplementedError("write your pallas kernel here")


# speedup vs baseline: 4.3040x; 4.3040x over previous
import functools

import jax
import jax.numpy as jnp
from jax import lax
from jax.experimental import pallas as pl
from jax.experimental.pallas import tpu as pltpu
from jax.experimental.pallas import tpu_sc as plsc


_BN = 400


def _mm2_body(h_ref, wr_ref, wroot_ref, broot_ref, hw_ref, root_ref):
    h = h_ref[...]
    hw_ref[...] = jnp.dot(h, wr_ref[...], preferred_element_type=jnp.float32)
    root_ref[...] = (
        jnp.dot(h, wroot_ref[...], preferred_element_type=jnp.float32)
        + broot_ref[...]
    )


def _mm2(h, Wr, Wroot, broot1d):
    n, d = h.shape
    grid = (n // _BN,)
    return pl.pallas_call(
        _mm2_body,
        grid=grid,
        in_specs=[
            pl.BlockSpec((_BN, d), lambda i: (i, 0)),
            pl.BlockSpec((d, d), lambda i: (0, 0)),
            pl.BlockSpec((d, d), lambda i: (0, 0)),
            pl.BlockSpec((1, d), lambda i: (0, 0)),
        ],
        out_specs=[
            pl.BlockSpec((_BN, d), lambda i: (i, 0)),
            pl.BlockSpec((_BN, d), lambda i: (i, 0)),
        ],
        out_shape=[
            jax.ShapeDtypeStruct((n, d), jnp.float32),
            jax.ShapeDtypeStruct((n, d), jnp.float32),
        ],
    )(h, Wr, Wroot, broot1d.reshape(1, d))


def _layer_post(root_ref, a_ref, ws_ref, g_ref, b_ref):
    agg = a_ref[0] + a_ref[1]
    ws = jnp.sum(ws_ref[...], axis=-1, keepdims=True)
    hidden = root_ref[...] + agg / jnp.maximum(ws, 1.0)
    mu = jnp.mean(hidden, axis=-1, keepdims=True)
    cen = hidden - mu
    var = jnp.mean(cen * cen, axis=-1, keepdims=True)
    hidden = cen * lax.rsqrt(var + 1e-5) * g_ref[...] + b_ref[...]
    return hidden * 0.5 * (1.0 + lax.erf(hidden * 0.7071067811865476))


def _comb_mm_body(root_ref, a_ref, ws_ref, g_ref, b_ref, wr_ref, wroot_ref,
                  broot_ref, hw_ref, root2_ref):
    h = _layer_post(root_ref, a_ref, ws_ref, g_ref, b_ref)
    hw_ref[...] = jnp.dot(h, wr_ref[...], preferred_element_type=jnp.float32)
    root2_ref[...] = (
        jnp.dot(h, wroot_ref[...], preferred_element_type=jnp.float32)
        + broot_ref[...]
    )


def _comb_mm(root, agg, ws, g1d, b1d, Wr, Wroot, broot1d):
    n, d = root.shape
    grid = (n // _BN,)
    return pl.pallas_call(
        _comb_mm_body,
        grid=grid,
        in_specs=[
            pl.BlockSpec((_BN, d), lambda i: (i, 0)),
            pl.BlockSpec((2, _BN, d), lambda i: (0, i, 0)),
            pl.BlockSpec((_BN, 32), lambda i: (i, 0)),
            pl.BlockSpec((1, d), lambda i: (0, 0)),
            pl.BlockSpec((1, d), lambda i: (0, 0)),
            pl.BlockSpec((d, d), lambda i: (0, 0)),
            pl.BlockSpec((d, d), lambda i: (0, 0)),
            pl.BlockSpec((1, d), lambda i: (0, 0)),
        ],
        out_specs=[
            pl.BlockSpec((_BN, d), lambda i: (i, 0)),
            pl.BlockSpec((_BN, d), lambda i: (i, 0)),
        ],
        out_shape=[
            jax.ShapeDtypeStruct((n, d), jnp.float32),
            jax.ShapeDtypeStruct((n, d), jnp.float32),
        ],
    )(root, agg, ws, g1d.reshape(1, d), b1d.reshape(1, d), Wr, Wroot,
      broot1d.reshape(1, d))


def _final_body(root_ref, a_ref, ws_ref, g_ref, b_ref, out_ref):
    out_ref[...] = _layer_post(root_ref, a_ref, ws_ref, g_ref, b_ref)


def _final(root, agg, ws, g1d, b1d):
    n, d = root.shape
    grid = (n // _BN,)
    return pl.pallas_call(
        _final_body,
        grid=grid,
        in_specs=[
            pl.BlockSpec((_BN, d), lambda i: (i, 0)),
            pl.BlockSpec((2, _BN, d), lambda i: (0, i, 0)),
            pl.BlockSpec((_BN, 32), lambda i: (i, 0)),
            pl.BlockSpec((1, d), lambda i: (0, 0)),
            pl.BlockSpec((1, d), lambda i: (0, 0)),
        ],
        out_specs=pl.BlockSpec((_BN, d), lambda i: (i, 0)),
        out_shape=jax.ShapeDtypeStruct((n, d), jnp.float32),
    )(root, agg, ws, g1d.reshape(1, d), b1d.reshape(1, d))


_NC = 2
_NS = 16
_C = 80
_WB = 80


def _edge_body(n, epw, nchunk,
               hw, src, dst, mask, agg_o, ws_o,
               agg_sh, src_v, dst_v, mask_v, rows_v, ws_local,
               zrow_v, sem):
    d = 128
    cid = lax.axis_index("c")
    sid = lax.axis_index("s")
    wid = cid * _NS + sid
    nb = n // _WB
    nbi = (nb + _NS - 1) // _NS

    def zb_body(i, _):
        for gg in range(d // 16):
            zrow_v[i, pl.ds(gg * 16, 16)] = jnp.zeros((16,), jnp.float32)
        return 0

    lax.fori_loop(0, _WB, zb_body, 0)

    def zw_body(i, _):
        ws_local[pl.ds(i * 16, 16)] = jnp.zeros((16,), jnp.float32)
        return 0

    lax.fori_loop(0, n // 16, zw_body, 0)

    def zero_body(i, _):
        b = i * _NS + sid

        @pl.when(b < nb)
        def _():
            pltpu.sync_copy(zrow_v, agg_sh.at[pl.ds(b * _WB, _WB)])

        return 0

    lax.fori_loop(0, nbi, zero_body, 0)
    plsc.subcore_barrier()

    ebase = wid * epw
    ii = lax.iota(jnp.int32, 16)

    def chunk_body(i, _):
        off = ebase + i * _C
        pltpu.sync_copy(src.at[pl.ds(off, _C)], src_v)
        pltpu.sync_copy(dst.at[pl.ds(off, _C)], dst_v)
        pltpu.sync_copy(mask.at[pl.ds(off, _C)], mask_v)
        pltpu.async_copy(hw.at[src_v], rows_v, sem).wait()

        def scale_body(eg, _):
            sl16 = pl.ds(eg * 16, 16)
            mvec = mask_v[sl16]
            dvec = dst_v[sl16]
            fs = mvec
            later = ii < 0
            base = eg * 16
            for k in range(1, 16):
                idx = base + ((ii + k) & 15)
                rd = plsc.load_gather(dst_v, [idx])
                rm = plsc.load_gather(mask_v, [idx])
                match = rd == dvec
                fs = fs + jnp.where(match, rm, 0.0)
                later = later | (match & (ii + k < 16))
            plsc.addupdate_scatter(ws_local, [dvec], fs,
                                   mask=jnp.logical_not(later))
            for ll in range(16):
                m = mvec[ll]
                e = eg * 16 + ll
                for gg in range(d // 16):
                    sl = pl.ds(gg * 16, 16)
                    rows_v[e, sl] = rows_v[e, sl] * m
            return 0

        lax.fori_loop(0, _C // 16, scale_body, 0)
        pltpu.sync_copy(rows_v, agg_sh.at[dst_v], add=True)
        return 0

    lax.fori_loop(0, nchunk, chunk_body, 0)
    plsc.subcore_barrier()

    pltpu.sync_copy(ws_local, ws_o.at[wid, 0])

    def wb_body(i, _):
        b = i * _NS + sid

        @pl.when(b < nb)
        def _():
            s0 = b * _WB
            pltpu.sync_copy(agg_sh.at[pl.ds(s0, _WB)], zrow_v)
            pltpu.sync_copy(zrow_v, agg_o.at[cid, pl.ds(s0, _WB)])

        return 0

    lax.fori_loop(0, nbi, wb_body, 0)


def _edge(hw, src, dst, mask):
    n, d = hw.shape
    e = src.shape[0]
    nw = _NC * _NS
    epw = e // nw
    nchunk = epw // _C
    mesh = plsc.VectorSubcoreMesh(
        core_axis_name="c", subcore_axis_name="s",
        num_cores=_NC, num_subcores=_NS)
    body = functools.partial(_edge_body, n, epw, nchunk)
    f = pl.kernel(
        body,
        out_type=[
            jax.ShapeDtypeStruct((_NC, n, d), jnp.float32),
            jax.ShapeDtypeStruct((_NC * _NS, 1, n), jnp.float32),
        ],
        mesh=mesh,
        scratch_types=[
            pltpu.VMEM_SHARED((n, d), jnp.float32),
            pltpu.VMEM((_C,), jnp.int32),
            pltpu.VMEM((_C,), jnp.int32),
            pltpu.VMEM((_C,), jnp.float32),
            pltpu.VMEM((_C, d), jnp.float32),
            pltpu.VMEM((n,), jnp.float32),
            pltpu.VMEM((_WB, d), jnp.float32),
            pltpu.SemaphoreType.DMA,
        ],
        compiler_params=pltpu.CompilerParams(needs_layout_passes=False),
    )
    agg, ws = f(hw, src, dst, mask)
    return agg, jnp.transpose(ws.reshape(_NC * _NS, n), (1, 0))


def kernel(x, edge_index, edge_mask, Wr0, Wroot0, broot0, g0, b0,
           Wr1, Wroot1, broot1, g1, b1):
    src = edge_index[0].astype(jnp.int32)
    dst = edge_index[1].astype(jnp.int32)
    mask = edge_mask.astype(jnp.float32)

    hw0, root0 = _mm2(x, Wr0, Wroot0, broot0)
    agg0, ws0 = _edge(hw0, src, dst, mask)
    hw1, root1 = _comb_mm(root0, agg0, ws0, g0, b0, Wr1, Wroot1, broot1)
    agg1, ws1 = _edge(hw1, src, dst, mask)
    return _final(root1, agg1, ws1, g1, b1)


# speedup vs baseline: 8.6629x; 2.0127x over previous
import functools

import jax
import jax.numpy as jnp
from jax import lax
from jax.experimental import pallas as pl
from jax.experimental.pallas import tpu as pltpu
from jax.experimental.pallas import tpu_sc as plsc


_BN = 400


def _mm2_body(h_ref, wr_ref, wroot_ref, broot_ref, hw_ref, root_ref):
    h = h_ref[...]
    hw_ref[...] = jnp.dot(h, wr_ref[...], preferred_element_type=jnp.float32)
    root_ref[...] = (
        jnp.dot(h, wroot_ref[...], preferred_element_type=jnp.float32)
        + broot_ref[...]
    )


def _mm2(h, Wr, Wroot, broot1d):
    n, d = h.shape
    grid = (n // _BN,)
    return pl.pallas_call(
        _mm2_body,
        grid=grid,
        in_specs=[
            pl.BlockSpec((_BN, d), lambda i: (i, 0)),
            pl.BlockSpec((d, d), lambda i: (0, 0)),
            pl.BlockSpec((d, d), lambda i: (0, 0)),
            pl.BlockSpec((1, d), lambda i: (0, 0)),
        ],
        out_specs=[
            pl.BlockSpec((_BN, d), lambda i: (i, 0)),
            pl.BlockSpec((_BN, d), lambda i: (i, 0)),
        ],
        out_shape=[
            jax.ShapeDtypeStruct((n, d), jnp.float32),
            jax.ShapeDtypeStruct((n, d), jnp.float32),
        ],
    )(h, Wr, Wroot, broot1d.reshape(1, d))


def _layer_post(root_ref, a_ref, ws_ref, g_ref, b_ref):
    agg = a_ref[0] + a_ref[1]
    ws = jnp.sum(ws_ref[...], axis=-1, keepdims=True)
    hidden = root_ref[...] + agg / jnp.maximum(ws, 1.0)
    mu = jnp.mean(hidden, axis=-1, keepdims=True)
    cen = hidden - mu
    var = jnp.mean(cen * cen, axis=-1, keepdims=True)
    hidden = cen * lax.rsqrt(var + 1e-5) * g_ref[...] + b_ref[...]
    return hidden * 0.5 * (1.0 + lax.erf(hidden * 0.7071067811865476))


def _comb_mm_body(root_ref, a_ref, ws_ref, g_ref, b_ref, wr_ref, wroot_ref,
                  broot_ref, hw_ref, root2_ref):
    h = _layer_post(root_ref, a_ref, ws_ref, g_ref, b_ref)
    hw_ref[...] = jnp.dot(h, wr_ref[...], preferred_element_type=jnp.float32)
    root2_ref[...] = (
        jnp.dot(h, wroot_ref[...], preferred_element_type=jnp.float32)
        + broot_ref[...]
    )


def _comb_mm(root, agg, ws, g1d, b1d, Wr, Wroot, broot1d):
    n, d = root.shape
    grid = (n // _BN,)
    return pl.pallas_call(
        _comb_mm_body,
        grid=grid,
        in_specs=[
            pl.BlockSpec((_BN, d), lambda i: (i, 0)),
            pl.BlockSpec((2, _BN, d), lambda i: (0, i, 0)),
            pl.BlockSpec((_BN, 32), lambda i: (i, 0)),
            pl.BlockSpec((1, d), lambda i: (0, 0)),
            pl.BlockSpec((1, d), lambda i: (0, 0)),
            pl.BlockSpec((d, d), lambda i: (0, 0)),
            pl.BlockSpec((d, d), lambda i: (0, 0)),
            pl.BlockSpec((1, d), lambda i: (0, 0)),
        ],
        out_specs=[
            pl.BlockSpec((_BN, d), lambda i: (i, 0)),
            pl.BlockSpec((_BN, d), lambda i: (i, 0)),
        ],
        out_shape=[
            jax.ShapeDtypeStruct((n, d), jnp.float32),
            jax.ShapeDtypeStruct((n, d), jnp.float32),
        ],
    )(root, agg, ws, g1d.reshape(1, d), b1d.reshape(1, d), Wr, Wroot,
      broot1d.reshape(1, d))


def _final_body(root_ref, a_ref, ws_ref, g_ref, b_ref, out_ref):
    out_ref[...] = _layer_post(root_ref, a_ref, ws_ref, g_ref, b_ref)


def _final(root, agg, ws, g1d, b1d):
    n, d = root.shape
    grid = (n // _BN,)
    return pl.pallas_call(
        _final_body,
        grid=grid,
        in_specs=[
            pl.BlockSpec((_BN, d), lambda i: (i, 0)),
            pl.BlockSpec((2, _BN, d), lambda i: (0, i, 0)),
            pl.BlockSpec((_BN, 32), lambda i: (i, 0)),
            pl.BlockSpec((1, d), lambda i: (0, 0)),
            pl.BlockSpec((1, d), lambda i: (0, 0)),
        ],
        out_specs=pl.BlockSpec((_BN, d), lambda i: (i, 0)),
        out_shape=jax.ShapeDtypeStruct((n, d), jnp.float32),
    )(root, agg, ws, g1d.reshape(1, d), b1d.reshape(1, d))


_NC = 2
_NS = 16
_C = 128
_WB = 80


def _edge_body(n, epw, nchunk, rem,
               hw, src3, dst3, mask3, tsrc, tdst, tmask, agg_o, ws_o,
               agg_sh, src_v0, src_v1, dst_v0, dst_v1, mask_v0, mask_v1,
               rows_v, ws_local, tsrc_v, tdst_v, tmask_v,
               sem0, sem1, semi0, semi1):
    d = 128
    cid = lax.axis_index("c")
    sid = lax.axis_index("s")
    wid = cid * _NS + sid
    nb = n // _WB
    nbi = (nb + _NS - 1) // _NS

    def zb_body(i, _):
        for gg in range(d // 16):
            rows_v[0, i, pl.ds(gg * 16, 16)] = jnp.zeros((16,), jnp.float32)
        return 0

    lax.fori_loop(0, _WB, zb_body, 0)

    def zw_body(i, _):
        ws_local[pl.ds(i * 16, 16)] = jnp.zeros((16,), jnp.float32)
        return 0

    lax.fori_loop(0, n // 16, zw_body, 0)

    def zero_body(i, _):
        b = i * _NS + sid

        @pl.when(b < nb)
        def _():
            pltpu.sync_copy(rows_v.at[0, pl.ds(0, _WB)],
                            agg_sh.at[pl.ds(b * _WB, _WB)])

        return 0

    lax.fori_loop(0, nbi, zero_body, 0)
    plsc.subcore_barrier()

    ii = lax.iota(jnp.int32, 16)
    gsems = (sem0, sem1)
    isems = (semi0, semi1)
    srcb = (src_v0, src_v1)
    dstb = (dst_v0, dst_v1)
    maskb = (mask_v0, mask_v1)

    def idx_start(i, par):
        pltpu.async_copy(src3.at[wid, i], srcb[par], isems[par])
        pltpu.async_copy(dst3.at[wid, i], dstb[par], isems[par])
        pltpu.async_copy(mask3.at[wid, i], maskb[par], isems[par])

    def idx_wait(i, par):
        pltpu.make_async_copy(src3.at[wid, i], srcb[par], isems[par]).wait()
        pltpu.make_async_copy(dst3.at[wid, i], dstb[par], isems[par]).wait()
        pltpu.make_async_copy(mask3.at[wid, i], maskb[par],
                              isems[par]).wait()

    def gather_start(i, b):
        pltpu.async_copy(hw.at[srcb[b]], rows_v.at[b], gsems[b])

    def gather_wait(i, b):
        pltpu.make_async_copy(hw.at[srcb[b]], rows_v.at[b], gsems[b]).wait()

    def scale_and_scatter(i, par):
        dst_v = dstb[par]
        mask_v = maskb[par]

        def scale_body(eg, _):
            sl16 = pl.ds(eg * 16, 16)
            dvec = dst_v[sl16]
            mvec = mask_v[sl16]
            fs = mvec
            later = ii < 0
            for k in range(1, 16):
                idx1 = eg * 16 + ((ii + k) & 15)
                rd = plsc.load_gather(dst_v, [idx1])
                rm = plsc.load_gather(mask_v, [idx1])
                match = rd == dvec
                fs = fs + jnp.where(match, rm, 0.0)
                later = later | (match & (ii + k < 16))
            plsc.addupdate_scatter(ws_local, [dvec], fs,
                                   mask=jnp.logical_not(later))
            for ll in range(16):
                m = mvec[ll]
                e = eg * 16 + ll
                for gg in range(d // 16):
                    sl = pl.ds(gg * 16, 16)
                    rows_v[par, e, sl] = rows_v[par, e, sl] * m
            return 0

        lax.fori_loop(0, _C // 16, scale_body, 0)
        pltpu.sync_copy(rows_v.at[par], agg_sh.at[dst_v], add=True)

    idx_start(0, 0)
    idx_wait(0, 0)
    gather_start(0, 0)
    idx_start(1, 1)

    def pair_body(j, _):
        i0 = j * 2
        gather_wait(i0, 0)
        idx_wait(i0 + 1, 1)
        gather_start(i0 + 1, 1)
        scale_and_scatter(i0, 0)

        @pl.when(i0 + 2 < nchunk)
        def _():
            idx_start(i0 + 2, 0)
            idx_wait(i0 + 2, 0)
            gather_wait(i0 + 1, 1)
            gather_start(i0 + 2, 0)

        @pl.when(i0 + 2 >= nchunk)
        def _():
            gather_wait(i0 + 1, 1)

        scale_and_scatter(i0 + 1, 1)

        @pl.when(i0 + 3 < nchunk)
        def _():
            idx_start(i0 + 3, 1)

        return 0

    lax.fori_loop(0, nchunk // 2, pair_body, 0)

    if rem:
        pltpu.sync_copy(tsrc.at[wid], tsrc_v)
        pltpu.sync_copy(tdst.at[wid], tdst_v)
        pltpu.sync_copy(tmask.at[wid], tmask_v)
        pltpu.async_copy(hw.at[tsrc_v.at[0]],
                         rows_v.at[0, pl.ds(0, rem)], sem0).wait()
        zerov = jnp.broadcast_to(jnp.int32(0), (16,))
        dvec = tdst_v[0, :]
        mvec = tmask_v[0, :]
        fs = mvec
        later = ii < 0
        for k in range(1, 16):
            idx1 = (ii + k) & 15
            rd = plsc.load_gather(tdst_v, [zerov, idx1])
            rm = plsc.load_gather(tmask_v, [zerov, idx1])
            match = rd == dvec
            fs = fs + jnp.where(match, rm, 0.0)
            later = later | (match & (ii + k < 16))
        plsc.addupdate_scatter(ws_local, [dvec], fs,
                               mask=jnp.logical_not(later))
        for ll in range(16):
            m = mvec[ll]
            for gg in range(d // 16):
                sl = pl.ds(gg * 16, 16)
                rows_v[0, ll, sl] = rows_v[0, ll, sl] * m
        pltpu.sync_copy(rows_v.at[0, pl.ds(0, rem)],
                        agg_sh.at[tdst_v.at[0]], add=True)
    plsc.subcore_barrier()

    pltpu.sync_copy(ws_local, ws_o.at[wid, 0])

    def wb_body(i, _):
        b = i * _NS + sid

        @pl.when(b < nb)
        def _():
            s0 = b * _WB
            pltpu.sync_copy(agg_sh.at[pl.ds(s0, _WB)],
                            rows_v.at[0, pl.ds(0, _WB)])
            pltpu.sync_copy(rows_v.at[0, pl.ds(0, _WB)],
                            agg_o.at[cid, pl.ds(s0, _WB)])

        return 0

    lax.fori_loop(0, nbi, wb_body, 0)


def _edge(hw, src, dst, mask):
    n, d = hw.shape
    e = src.shape[0]
    nw = _NC * _NS
    epw = e // nw
    nchunk = epw // _C
    rem = epw - nchunk * _C
    assert rem in (0, 16) and nchunk % 2 == 0
    mesh = plsc.VectorSubcoreMesh(
        core_axis_name="c", subcore_axis_name="s",
        num_cores=_NC, num_subcores=_NS)
    body = functools.partial(_edge_body, n, epw, nchunk, rem)
    f = pl.kernel(
        body,
        out_type=[
            jax.ShapeDtypeStruct((_NC, n, d), jnp.float32),
            jax.ShapeDtypeStruct((_NC * _NS, 1, n), jnp.float32),
        ],
        mesh=mesh,
        scratch_types=[
            pltpu.VMEM_SHARED((n, d), jnp.float32),
            pltpu.VMEM((_C,), jnp.int32),
            pltpu.VMEM((_C,), jnp.int32),
            pltpu.VMEM((_C,), jnp.int32),
            pltpu.VMEM((_C,), jnp.int32),
            pltpu.VMEM((_C,), jnp.float32),
            pltpu.VMEM((_C,), jnp.float32),
            pltpu.VMEM((2, _C, d), jnp.float32),
            pltpu.VMEM((n,), jnp.float32),
            pltpu.VMEM((1, 16), jnp.int32),
            pltpu.VMEM((1, 16), jnp.int32),
            pltpu.VMEM((1, 16), jnp.float32),
            pltpu.SemaphoreType.DMA,
            pltpu.SemaphoreType.DMA,
            pltpu.SemaphoreType.DMA,
            pltpu.SemaphoreType.DMA,
        ],
        compiler_params=pltpu.CompilerParams(needs_layout_passes=False),
    )
    full = nchunk * _C
    src2 = src.reshape(nw, epw)
    dst2 = dst.reshape(nw, epw)
    mask2 = mask.reshape(nw, epw)
    src3 = src2[:, :full].reshape(nw, nchunk, _C)
    dst3 = dst2[:, :full].reshape(nw, nchunk, _C)
    mask3 = mask2[:, :full].reshape(nw, nchunk, _C)
    tsrc = src2[:, full:].reshape(nw, 1, rem)
    tdst = dst2[:, full:].reshape(nw, 1, rem)
    tmask = mask2[:, full:].reshape(nw, 1, rem)
    agg, ws = f(hw, src3, dst3, mask3, tsrc, tdst, tmask)
    return agg, jnp.transpose(ws.reshape(_NC * _NS, n), (1, 0))


def kernel(x, edge_index, edge_mask, Wr0, Wroot0, broot0, g0, b0,
           Wr1, Wroot1, broot1, g1, b1):
    src = edge_index[0].astype(jnp.int32)
    dst = edge_index[1].astype(jnp.int32)
    mask = edge_mask.astype(jnp.float32)

    hw0, root0 = _mm2(x, Wr0, Wroot0, broot0)
    agg0, ws0 = _edge(hw0, src, dst, mask)
    hw1, root1 = _comb_mm(root0, agg0, ws0, g0, b0, Wr1, Wroot1, broot1)
    agg1, ws1 = _edge(hw1, src, dst, mask)
    return _final(root1, agg1, ws1, g1, b1)


# speedup vs baseline: 9.5609x; 1.1037x over previous
import functools

import jax
import jax.numpy as jnp
from jax import lax
from jax.experimental import pallas as pl
from jax.experimental.pallas import tpu as pltpu
from jax.experimental.pallas import tpu_sc as plsc


_BN = 400


def _mm2_body(h_ref, wr_ref, wroot_ref, broot_ref, hw_ref, root_ref):
    h = h_ref[...]
    hw_ref[...] = jnp.dot(h, wr_ref[...], preferred_element_type=jnp.float32)
    root_ref[...] = (
        jnp.dot(h, wroot_ref[...], preferred_element_type=jnp.float32)
        + broot_ref[...]
    )


def _mm2(h, Wr, Wroot, broot1d):
    n, d = h.shape
    grid = (n // _BN,)
    return pl.pallas_call(
        _mm2_body,
        grid=grid,
        in_specs=[
            pl.BlockSpec((_BN, d), lambda i: (i, 0)),
            pl.BlockSpec((d, d), lambda i: (0, 0)),
            pl.BlockSpec((d, d), lambda i: (0, 0)),
            pl.BlockSpec((1, d), lambda i: (0, 0)),
        ],
        out_specs=[
            pl.BlockSpec((_BN, d), lambda i: (i, 0)),
            pl.BlockSpec((_BN, d), lambda i: (i, 0)),
        ],
        out_shape=[
            jax.ShapeDtypeStruct((n, d), jnp.float32),
            jax.ShapeDtypeStruct((n, d), jnp.float32),
        ],
    )(h, Wr, Wroot, broot1d.reshape(1, d))


def _layer_post(root_ref, a_ref, ws_ref, g_ref, b_ref):
    agg = a_ref[0] + a_ref[1]
    ws = jnp.sum(ws_ref[...], axis=-1, keepdims=True)
    hidden = root_ref[...] + agg / jnp.maximum(ws, 1.0)
    mu = jnp.mean(hidden, axis=-1, keepdims=True)
    cen = hidden - mu
    var = jnp.mean(cen * cen, axis=-1, keepdims=True)
    hidden = cen * lax.rsqrt(var + 1e-5) * g_ref[...] + b_ref[...]
    return hidden * 0.5 * (1.0 + lax.erf(hidden * 0.7071067811865476))


def _comb_mm_body(root_ref, a_ref, ws_ref, g_ref, b_ref, wr_ref, wroot_ref,
                  broot_ref, hw_ref, root2_ref):
    h = _layer_post(root_ref, a_ref, ws_ref, g_ref, b_ref)
    hw_ref[...] = jnp.dot(h, wr_ref[...], preferred_element_type=jnp.float32)
    root2_ref[...] = (
        jnp.dot(h, wroot_ref[...], preferred_element_type=jnp.float32)
        + broot_ref[...]
    )


def _comb_mm(root, agg, ws, g1d, b1d, Wr, Wroot, broot1d):
    n, d = root.shape
    grid = (n // _BN,)
    return pl.pallas_call(
        _comb_mm_body,
        grid=grid,
        in_specs=[
            pl.BlockSpec((_BN, d), lambda i: (i, 0)),
            pl.BlockSpec((2, _BN, d), lambda i: (0, i, 0)),
            pl.BlockSpec((_BN, 32), lambda i: (i, 0)),
            pl.BlockSpec((1, d), lambda i: (0, 0)),
            pl.BlockSpec((1, d), lambda i: (0, 0)),
            pl.BlockSpec((d, d), lambda i: (0, 0)),
            pl.BlockSpec((d, d), lambda i: (0, 0)),
            pl.BlockSpec((1, d), lambda i: (0, 0)),
        ],
        out_specs=[
            pl.BlockSpec((_BN, d), lambda i: (i, 0)),
            pl.BlockSpec((_BN, d), lambda i: (i, 0)),
        ],
        out_shape=[
            jax.ShapeDtypeStruct((n, d), jnp.float32),
            jax.ShapeDtypeStruct((n, d), jnp.float32),
        ],
    )(root, agg, ws, g1d.reshape(1, d), b1d.reshape(1, d), Wr, Wroot,
      broot1d.reshape(1, d))


def _final_body(root_ref, a_ref, ws_ref, g_ref, b_ref, out_ref):
    out_ref[...] = _layer_post(root_ref, a_ref, ws_ref, g_ref, b_ref)


def _final(root, agg, ws, g1d, b1d):
    n, d = root.shape
    grid = (n // _BN,)
    return pl.pallas_call(
        _final_body,
        grid=grid,
        in_specs=[
            pl.BlockSpec((_BN, d), lambda i: (i, 0)),
            pl.BlockSpec((2, _BN, d), lambda i: (0, i, 0)),
            pl.BlockSpec((_BN, 32), lambda i: (i, 0)),
            pl.BlockSpec((1, d), lambda i: (0, 0)),
            pl.BlockSpec((1, d), lambda i: (0, 0)),
        ],
        out_specs=pl.BlockSpec((_BN, d), lambda i: (i, 0)),
        out_shape=jax.ShapeDtypeStruct((n, d), jnp.float32),
    )(root, agg, ws, g1d.reshape(1, d), b1d.reshape(1, d))


_NC = 2
_NS = 16
_C = 80
_NU = 3
_WB = 80


def _edge_body(n, epw, nchunk, rem,
               hw, src3, dst3, mask3, tsrc, tdst, tmask, agg_o, ws_o,
               agg_sh, src_v0, src_v1, src_v2, dst_v0, dst_v1, dst_v2,
               mask_v0, mask_v1, mask_v2,
               rows_v, ws_local, tsrc_v, tdst_v, tmask_v,
               sem0, sem1, sem2, semi0, semi1, semi2,
               sems0, sems1, sems2):
    d = 128
    cid = lax.axis_index("c")
    sid = lax.axis_index("s")
    wid = cid * _NS + sid
    nb = n // _WB
    nbi = (nb + _NS - 1) // _NS

    def zb_body(i, _):
        for gg in range(d // 16):
            rows_v[0, i, pl.ds(gg * 16, 16)] = jnp.zeros((16,), jnp.float32)
        return 0

    lax.fori_loop(0, _WB, zb_body, 0)

    def zw_body(i, _):
        ws_local[pl.ds(i * 16, 16)] = jnp.zeros((16,), jnp.float32)
        return 0

    lax.fori_loop(0, n // 16, zw_body, 0)

    def zero_body(i, _):
        b = i * _NS + sid

        @pl.when(b < nb)
        def _():
            pltpu.sync_copy(rows_v.at[0, pl.ds(0, _WB)],
                            agg_sh.at[pl.ds(b * _WB, _WB)])

        return 0

    lax.fori_loop(0, nbi, zero_body, 0)
    plsc.subcore_barrier()

    ii = lax.iota(jnp.int32, 16)
    gsems = (sem0, sem1, sem2)
    isems = (semi0, semi1, semi2)
    ssems = (sems0, sems1, sems2)
    srcb = (src_v0, src_v1, src_v2)
    dstb = (dst_v0, dst_v1, dst_v2)
    maskb = (mask_v0, mask_v1, mask_v2)

    def idx_start(i, par):
        pltpu.async_copy(src3.at[wid, i], srcb[par], isems[par])
        pltpu.async_copy(dst3.at[wid, i], dstb[par], isems[par])
        pltpu.async_copy(mask3.at[wid, i], maskb[par], isems[par])

    def idx_wait(i, par):
        pltpu.make_async_copy(src3.at[wid, i], srcb[par], isems[par]).wait()
        pltpu.make_async_copy(dst3.at[wid, i], dstb[par], isems[par]).wait()
        pltpu.make_async_copy(mask3.at[wid, i], maskb[par],
                              isems[par]).wait()

    def gather_start(i, b):
        pltpu.async_copy(hw.at[srcb[b]], rows_v.at[b], gsems[b])

    def gather_wait(i, b):
        pltpu.make_async_copy(hw.at[srcb[b]], rows_v.at[b], gsems[b]).wait()

    def scale_and_scatter(i, par):
        dst_v = dstb[par]
        mask_v = maskb[par]

        def scale_body(eg, _):
            sl16 = pl.ds(eg * 16, 16)
            dvec = dst_v[sl16]
            mvec = mask_v[sl16]
            fs = mvec
            later = ii < 0
            for k in range(1, 16):
                idx1 = eg * 16 + ((ii + k) & 15)
                rd = plsc.load_gather(dst_v, [idx1])
                rm = plsc.load_gather(mask_v, [idx1])
                match = rd == dvec
                fs = fs + jnp.where(match, rm, 0.0)
                later = later | (match & (ii + k < 16))
            plsc.addupdate_scatter(ws_local, [dvec], fs,
                                   mask=jnp.logical_not(later))
            for ll in range(16):
                m = mvec[ll]
                e = eg * 16 + ll
                for gg in range(d // 16):
                    sl = pl.ds(gg * 16, 16)
                    rows_v[par, e, sl] = rows_v[par, e, sl] * m
            return 0

        lax.fori_loop(0, _C // 16, scale_body, 0)

    def scatter_start(i, par):
        pltpu.async_copy(rows_v.at[par], agg_sh.at[dstb[par]], ssems[par],
                         add=True)

    def scatter_wait(i, par):
        pltpu.make_async_copy(rows_v.at[par], agg_sh.at[dstb[par]],
                              ssems[par]).wait()

    for u in range(_NU - 1):
        idx_start(u, u)
        idx_wait(u, u)
        gather_start(u, u)

    def turn(i, u):
        gather_wait(i, u)
        scale_and_scatter(i, u)
        scatter_start(i, u)

        @pl.when(i + 2 < nchunk)
        def _():
            u2 = (u + 2) % _NU

            @pl.when(i >= 1)
            def _():
                scatter_wait(i - 1, u2)

            idx_start(i + 2, u2)
            idx_wait(i + 2, u2)
            gather_start(i + 2, u2)

    def chunk_body(i, _):
        u = lax.rem(i, _NU)
        for uu in range(_NU):
            pl.when(u == uu)(lambda uu=uu: turn(i, uu))
        return 0

    lax.fori_loop(0, nchunk, chunk_body, 0)
    for tail_i in range(nchunk - _NU, nchunk):
        scatter_wait(tail_i, tail_i % _NU)

    if rem:
        pltpu.sync_copy(tsrc.at[wid], tsrc_v)
        pltpu.sync_copy(tdst.at[wid], tdst_v)
        pltpu.sync_copy(tmask.at[wid], tmask_v)
        pltpu.async_copy(hw.at[tsrc_v.at[0]],
                         rows_v.at[0, pl.ds(0, rem)], sem0).wait()
        zerov = jnp.broadcast_to(jnp.int32(0), (16,))
        dvec = tdst_v[0, :]
        mvec = tmask_v[0, :]
        fs = mvec
        later = ii < 0
        for k in range(1, 16):
            idx1 = (ii + k) & 15
            rd = plsc.load_gather(tdst_v, [zerov, idx1])
            rm = plsc.load_gather(tmask_v, [zerov, idx1])
            match = rd == dvec
            fs = fs + jnp.where(match, rm, 0.0)
            later = later | (match & (ii + k < 16))
        plsc.addupdate_scatter(ws_local, [dvec], fs,
                               mask=jnp.logical_not(later))
        for ll in range(16):
            m = mvec[ll]
            for gg in range(d // 16):
                sl = pl.ds(gg * 16, 16)
                rows_v[0, ll, sl] = rows_v[0, ll, sl] * m
        pltpu.sync_copy(rows_v.at[0, pl.ds(0, rem)],
                        agg_sh.at[tdst_v.at[0]], add=True)
    plsc.subcore_barrier()

    pltpu.sync_copy(ws_local, ws_o.at[wid, 0])

    def wb_body(i, _):
        b = i * _NS + sid

        @pl.when(b < nb)
        def _():
            s0 = b * _WB
            pltpu.sync_copy(agg_sh.at[pl.ds(s0, _WB)],
                            rows_v.at[0, pl.ds(0, _WB)])
            pltpu.sync_copy(rows_v.at[0, pl.ds(0, _WB)],
                            agg_o.at[cid, pl.ds(s0, _WB)])

        return 0

    lax.fori_loop(0, nbi, wb_body, 0)


def _edge(hw, src, dst, mask):
    n, d = hw.shape
    e = src.shape[0]
    nw = _NC * _NS
    epw = e // nw
    nchunk = epw // _C
    rem = epw - nchunk * _C
    assert rem in (0, 16) and nchunk > _NU
    mesh = plsc.VectorSubcoreMesh(
        core_axis_name="c", subcore_axis_name="s",
        num_cores=_NC, num_subcores=_NS)
    body = functools.partial(_edge_body, n, epw, nchunk, rem)
    f = pl.kernel(
        body,
        out_type=[
            jax.ShapeDtypeStruct((_NC, n, d), jnp.float32),
            jax.ShapeDtypeStruct((_NC * _NS, 1, n), jnp.float32),
        ],
        mesh=mesh,
        scratch_types=[
            pltpu.VMEM_SHARED((n, d), jnp.float32),
            pltpu.VMEM((_C,), jnp.int32),
            pltpu.VMEM((_C,), jnp.int32),
            pltpu.VMEM((_C,), jnp.int32),
            pltpu.VMEM((_C,), jnp.int32),
            pltpu.VMEM((_C,), jnp.int32),
            pltpu.VMEM((_C,), jnp.int32),
            pltpu.VMEM((_C,), jnp.float32),
            pltpu.VMEM((_C,), jnp.float32),
            pltpu.VMEM((_C,), jnp.float32),
            pltpu.VMEM((_NU, _C, d), jnp.float32),
            pltpu.VMEM((n,), jnp.float32),
            pltpu.VMEM((1, 16), jnp.int32),
            pltpu.VMEM((1, 16), jnp.int32),
            pltpu.VMEM((1, 16), jnp.float32),
        ] + [pltpu.SemaphoreType.DMA] * 9,
        compiler_params=pltpu.CompilerParams(needs_layout_passes=False),
    )
    full = nchunk * _C
    src2 = src.reshape(nw, epw)
    dst2 = dst.reshape(nw, epw)
    mask2 = mask.reshape(nw, epw)
    src3 = src2[:, :full].reshape(nw, nchunk, _C)
    dst3 = dst2[:, :full].reshape(nw, nchunk, _C)
    mask3 = mask2[:, :full].reshape(nw, nchunk, _C)
    if rem:
        tsrc = src2[:, full:].reshape(nw, 1, rem)
        tdst = dst2[:, full:].reshape(nw, 1, rem)
        tmask = mask2[:, full:].reshape(nw, 1, rem)
    else:
        tsrc = jnp.zeros((nw, 1, 16), jnp.int32)
        tdst = jnp.zeros((nw, 1, 16), jnp.int32)
        tmask = jnp.zeros((nw, 1, 16), jnp.float32)
    agg, ws = f(hw, src3, dst3, mask3, tsrc, tdst, tmask)
    return agg, jnp.transpose(ws.reshape(_NC * _NS, n), (1, 0))


def kernel(x, edge_index, edge_mask, Wr0, Wroot0, broot0, g0, b0,
           Wr1, Wroot1, broot1, g1, b1):
    src = edge_index[0].astype(jnp.int32)
    dst = edge_index[1].astype(jnp.int32)
    mask = edge_mask.astype(jnp.float32)

    hw0, root0 = _mm2(x, Wr0, Wroot0, broot0)
    agg0, ws0 = _edge(hw0, src, dst, mask)
    hw1, root1 = _comb_mm(root0, agg0, ws0, g0, b0, Wr1, Wroot1, broot1)
    agg1, ws1 = _edge(hw1, src, dst, mask)
    return _final(root1, agg1, ws1, g1, b1)
